# Initial kernel scaffold; baseline (speedup 1.0000x reference)
import jax
import jax.numpy as jnp
from jax.experimental import pallas as pl


def kernel(x, edge_index, batch, W0, b0, Wc, bc, Wih, bih, Whh, bhh, W1, b1, W2, b2):
    raise NotImplementedError("write your pallas kernel here")


# speedup vs baseline: 6.4869x; 6.4869x over previous
import functools

import jax
import jax.numpy as jnp
from jax import lax
from jax.experimental import pallas as pl
from jax.experimental.pallas import tpu as pltpu
from jax.experimental.pallas import tpu_sc as plsc

_N = 50000
_E = 800000
_DIN = 128
_DIM = 64
_OUT = 2
_B = 256

_NC = 2
_NS = 16
_NW = _NC * _NS
_E_PAD = 819200
_EPW = _E_PAD // _NW
_CH = 128
_NCH = _EPW // _CH

_DEGN = 51200
_DPT = _DEGN // _NS

_NSP = 3
_SPAN = 18432
_NT = _NSP * _SPAN
_ACCN = 18496
_APT = _ACCN // _NS
_TRASH = 18432

_mesh = plsc.VectorSubcoreMesh(core_axis_name="c", subcore_axis_name="s")


@functools.partial(
    pl.kernel,
    mesh=_mesh,
    out_type=jax.ShapeDtypeStruct((_NC, _DEGN), jnp.float32),
    scratch_types=[
        pltpu.VMEM((_CH,), jnp.int32),
        pltpu.VMEM((_CH,), jnp.float32),
        pltpu.VMEM((_DPT,), jnp.float32),
        pltpu.VMEM_SHARED((_DEGN,), jnp.float32),
    ],
)
def _sc_deg(dst_hbm, deg_out, dst_v, ones_v, zbuf, deg_sh):
    cid = lax.axis_index("c")
    sid = lax.axis_index("s")
    wid = sid * _NC + cid

    def fill_ones(i, _):
        ones_v[pl.ds(i * 16, 16)] = jnp.full((16,), 1.0, jnp.float32)
        return 0
    lax.fori_loop(0, _CH // 16, fill_ones, 0)

    def fill_zero(i, _):
        zbuf[pl.ds(i * 16, 16)] = jnp.zeros((16,), jnp.float32)
        return 0
    lax.fori_loop(0, _DPT // 16, fill_zero, 0)

    pltpu.sync_copy(zbuf, deg_sh.at[pl.ds(sid * _DPT, _DPT)])
    plsc.subcore_barrier()

    base = wid * _EPW

    def chunk(j, _):
        pltpu.sync_copy(dst_hbm.at[pl.ds(base + j * _CH, _CH)], dst_v)
        pltpu.sync_copy(ones_v, deg_sh.at[dst_v], add=True)
        return 0
    lax.fori_loop(0, _NCH, chunk, 0)

    plsc.subcore_barrier()
    pltpu.sync_copy(deg_sh.at[pl.ds(sid * _DPT, _DPT)],
                    deg_out.at[cid, pl.ds(sid * _DPT, _DPT)])


@functools.partial(
    pl.kernel,
    mesh=_mesh,
    out_type=jax.ShapeDtypeStruct((_NSP, _NC, _ACCN, _DIM), jnp.float32),
    scratch_types=[
        pltpu.VMEM((_CH,), jnp.int32),
        pltpu.VMEM((_CH,), jnp.int32),
        pltpu.VMEM((_CH,), jnp.int32),
        pltpu.VMEM((_CH, _DIM), jnp.float32),
        pltpu.VMEM((_APT // 4, _DIM), jnp.float32),
        pltpu.SemaphoreType.DMA,
        pltpu.VMEM_SHARED((_ACCN, _DIM), jnp.float32),
    ],
    compiler_params=pltpu.CompilerParams(use_tc_tiling_on_sc=False),
)
def _sc_agg(src_hbm, dst_hbm, h2p_hbm, acc_out,
            src_v, dstr_v, dstl_v, rows_v, zb2, sem, acc_sh):
    cid = lax.axis_index("c")
    sid = lax.axis_index("s")
    wid = sid * _NC + cid
    base = wid * _EPW
    zrows = _APT // 4

    def fill_zero(i, _):
        r = i // 4
        c = (i % 4) * 16
        zb2[r, pl.ds(c, 16)] = jnp.zeros((16,), jnp.float32)
        return 0
    lax.fori_loop(0, zrows * 4, fill_zero, 0)

    for sp in range(_NSP):
        for k in range(4):
            pltpu.sync_copy(zb2, acc_sh.at[pl.ds(sid * _APT + k * zrows, zrows)])
        plsc.subcore_barrier()
        lo = sp * _SPAN

        def chunk(j, _):
            pltpu.sync_copy(src_hbm.at[pl.ds(base + j * _CH, _CH)], src_v)
            pltpu.sync_copy(dst_hbm.at[pl.ds(base + j * _CH, _CH)], dstr_v)

            def remap(k2, _2):
                d = dstr_v[pl.ds(k2 * 16, 16)]
                dl = d - lo
                ok = (dl >= 0) & (dl < _SPAN)
                dstl_v[pl.ds(k2 * 16, 16)] = jnp.where(ok, dl, _TRASH)
                return 0
            lax.fori_loop(0, _CH // 16, remap, 0)

            pltpu.async_copy(h2p_hbm.at[src_v], rows_v, sem).wait()
            pltpu.sync_copy(rows_v, acc_sh.at[dstl_v], add=True)
            return 0
        lax.fori_loop(0, _NCH, chunk, 0)

        plsc.subcore_barrier()
        pltpu.sync_copy(acc_sh.at[pl.ds(sid * _APT, _APT)],
                        acc_out.at[sp, cid, pl.ds(sid * _APT, _APT)])
        plsc.subcore_barrier()


def _tc1_body(x_ref, w0_ref, b0_ref, wc_ref, o_ref):
    h = jnp.dot(x_ref[...], w0_ref[...], preferred_element_type=jnp.float32)
    h = jnp.maximum(h + b0_ref[...][None, :], 0.0)
    o_ref[...] = jnp.dot(h, wc_ref[...], preferred_element_type=jnp.float32)


_tc1 = pl.pallas_call(
    _tc1_body,
    grid=(50,),
    in_specs=[
        pl.BlockSpec((1000, _DIN), lambda i: (i, 0)),
        pl.BlockSpec((_DIN, _DIM), lambda i: (0, 0)),
        pl.BlockSpec((_DIM,), lambda i: (0,)),
        pl.BlockSpec((_DIM, _DIM), lambda i: (0, 0)),
    ],
    out_specs=pl.BlockSpec((1000, _DIM), lambda i: (i, 0)),
    out_shape=jax.ShapeDtypeStruct((_N, _DIM), jnp.float32),
)


def _tc2_body(degp_ref, h2_ref, dinv_ref, h2p_ref):
    deg = degp_ref[0, :] + degp_ref[1, :] + 1.0
    dinv = lax.rsqrt(deg)
    dinv_ref[...] = dinv[:, None]
    h2p_ref[...] = h2_ref[...] * dinv[:, None]


_tc2 = pl.pallas_call(
    _tc2_body,
    grid=(49,),
    in_specs=[
        pl.BlockSpec((_NC, 1024), lambda i: (0, i)),
        pl.BlockSpec((1024, _DIM), lambda i: (i, 0)),
    ],
    out_specs=[
        pl.BlockSpec((1024, 1), lambda i: (i, 0)),
        pl.BlockSpec((1024, _DIM), lambda i: (i, 0)),
    ],
    out_shape=[
        jax.ShapeDtypeStruct((_N, 1), jnp.float32),
        jax.ShapeDtypeStruct((_N, _DIM), jnp.float32),
    ],
)


def _tc3_body(accs_ref, dinv_ref, h2_ref, bc_ref, o_ref):
    agg = accs_ref[0, 0] + accs_ref[0, 1]
    dinv = dinv_ref[...]
    out = jnp.maximum(
        dinv * (agg + dinv * h2_ref[...]) + bc_ref[...][None, :], 0.0)
    row = (pl.program_id(0) * 1152
           + lax.broadcasted_iota(jnp.int32, (1152, 1), 0))
    out = jnp.where(row < _N, out, 0.0)
    o_ref[...] = out.T


_tc3 = pl.pallas_call(
    _tc3_body,
    grid=(44,),
    in_specs=[
        pl.BlockSpec((1, _NC, 1152, _DIM), lambda i: (i // 16, 0, i % 16, 0)),
        pl.BlockSpec((1152, 1), lambda i: (i, 0)),
        pl.BlockSpec((1152, _DIM), lambda i: (i, 0)),
        pl.BlockSpec((_DIM,), lambda i: (0,)),
    ],
    out_specs=pl.BlockSpec((_DIM, 1152), lambda i: (0, i)),
    out_shape=jax.ShapeDtypeStruct((_DIM, _NT), jnp.float32),
)


_NB = 27
_BS = 2048


def _tc4_body(ht_ref, batch_ref, wih_ref, bih_ref, whh_ref, bhh_ref,
              w1_ref, b1_ref, w2_ref, b2_ref, o_ref):
    iota_g = lax.broadcasted_iota(jnp.int32, (_B, 1), 0)
    hs = jnp.zeros((_B, _DIM), jnp.float32)
    cs = jnp.zeros((_B, _DIM), jnp.float32)
    q_star = jnp.zeros((_B, 2 * _DIM), jnp.float32)

    for _ in range(3):
        gates = (jnp.dot(q_star, wih_ref[...], preferred_element_type=jnp.float32)
                 + bih_ref[...][None, :]
                 + jnp.dot(hs, whh_ref[...], preferred_element_type=jnp.float32)
                 + bhh_ref[...][None, :])
        i_g = jax.nn.sigmoid(gates[:, :_DIM])
        f_g = jax.nn.sigmoid(gates[:, _DIM:2 * _DIM])
        g_g = jnp.tanh(gates[:, 2 * _DIM:3 * _DIM])
        o_g = jax.nn.sigmoid(gates[:, 3 * _DIM:])
        cs = f_g * cs + i_g * g_g
        hs = o_g * jnp.tanh(cs)
        q = hs

        def pass1(nb, emax):
            bb = batch_ref[pl.ds(nb * _BS, _BS)]
            valid = (bb < _B)[None, :]
            hbt = jnp.where(valid, ht_ref[:, pl.ds(nb * _BS, _BS)], 0.0)
            obt = (iota_g == bb[None, :]).astype(jnp.float32)
            qbt = lax.dot_general(q, obt, (((0,), (0,)), ((), ())),
                                  preferred_element_type=jnp.float32)
            ebt = jnp.sum(hbt * qbt, axis=0, keepdims=True)
            m = jnp.max(jnp.where(obt > 0, ebt, -1e30), axis=1, keepdims=True)
            return jnp.maximum(emax, m)
        emax = lax.fori_loop(0, _NB, pass1,
                             jnp.full((_B, 1), -1e30, jnp.float32))

        def pass2(nb, carry):
            denom, racc = carry
            bb = batch_ref[pl.ds(nb * _BS, _BS)]
            valid = (bb < _B)[None, :]
            hbt = jnp.where(valid, ht_ref[:, pl.ds(nb * _BS, _BS)], 0.0)
            obt = (iota_g == bb[None, :]).astype(jnp.float32)
            qbt = lax.dot_general(q, obt, (((0,), (0,)), ((), ())),
                                  preferred_element_type=jnp.float32)
            ebt = jnp.sum(hbt * qbt, axis=0, keepdims=True)
            mg = lax.dot_general(emax, obt, (((0,), (0,)), ((), ())),
                                 preferred_element_type=jnp.float32)
            ab = jnp.where(obt > 0, jnp.exp(ebt - mg), 0.0)
            denom = denom + jnp.sum(ab, axis=1, keepdims=True)
            racc = racc + lax.dot_general(
                ab, hbt, (((1,), (1,)), ((), ())),
                preferred_element_type=jnp.float32)
            return denom, racc
        denom, racc = lax.fori_loop(
            0, _NB, pass2,
            (jnp.zeros((_B, 1), jnp.float32),
             jnp.zeros((_B, _DIM), jnp.float32)))

        r = jnp.where(denom > 0, racc / denom, 0.0)
        q_star = jnp.concatenate([q, r], axis=1)

    o1 = jnp.maximum(
        jnp.dot(q_star, w1_ref[...], preferred_element_type=jnp.float32)
        + b1_ref[...][None, :], 0.0)
    o_ref[...] = (jnp.dot(o1, w2_ref[...], preferred_element_type=jnp.float32)
                  + b2_ref[...][None, :])


_tc4 = pl.pallas_call(
    _tc4_body,
    out_shape=jax.ShapeDtypeStruct((_B, _OUT), jnp.float32),
)


def kernel(x, edge_index, batch, W0, b0, Wc, bc, Wih, bih, Whh, bhh,
           W1, b1, W2, b2):
    src = edge_index[0].astype(jnp.int32)
    dst = edge_index[1].astype(jnp.int32)
    pad = _E_PAD - _E
    src_p = jnp.concatenate([src, jnp.zeros((pad,), jnp.int32)])
    dst_p = jnp.concatenate([dst, jnp.full((pad,), _N, jnp.int32)])
    batch32 = jnp.concatenate([batch.astype(jnp.int32),
                           jnp.full((_NT - _N,), _B, jnp.int32)])

    h2 = _tc1(x, W0, b0, Wc)
    degp = _sc_deg(dst_p)
    dinv, h2p = _tc2(degp, h2)
    accs = _sc_agg(src_p, dst_p, h2p)
    out2 = _tc3(accs, dinv, h2, bc)
    return _tc4(out2, batch32, Wih, bih, Whh, bhh, W1, b1, W2, b2)


# speedup vs baseline: 8.4825x; 1.3076x over previous
import functools

import jax
import jax.numpy as jnp
from jax import lax
from jax.experimental import pallas as pl
from jax.experimental.pallas import tpu as pltpu
from jax.experimental.pallas import tpu_sc as plsc

_N = 50000
_E = 800000
_DIN = 128
_DIM = 64
_OUT = 2
_B = 256

_NC = 2
_NS = 16
_NW = _NC * _NS
_E_PAD = 819200
_EPW = _E_PAD // _NW
_CH = 128
_NCH = _EPW // _CH

_DEGN = 51200
_DPT = _DEGN // _NS

_NSP = 3
_SPAN = 18432
_NT = _NSP * _SPAN
_ACCN = 18496
_APT = _ACCN // _NS
_TRASH = 18432

_mesh = plsc.VectorSubcoreMesh(core_axis_name="c", subcore_axis_name="s")


@functools.partial(
    pl.kernel,
    mesh=_mesh,
    out_type=jax.ShapeDtypeStruct((_NC, _DEGN), jnp.float32),
    scratch_types=[
        pltpu.VMEM((_CH,), jnp.int32),
        pltpu.VMEM((_CH,), jnp.float32),
        pltpu.VMEM((_DPT,), jnp.float32),
        pltpu.VMEM_SHARED((_DEGN,), jnp.float32),
    ],
)
def _sc_deg(dst_hbm, deg_out, dst_v, ones_v, zbuf, deg_sh):
    cid = lax.axis_index("c")
    sid = lax.axis_index("s")
    wid = sid * _NC + cid

    def fill_ones(i, _):
        ones_v[pl.ds(i * 16, 16)] = jnp.full((16,), 1.0, jnp.float32)
        return 0
    lax.fori_loop(0, _CH // 16, fill_ones, 0)

    def fill_zero(i, _):
        zbuf[pl.ds(i * 16, 16)] = jnp.zeros((16,), jnp.float32)
        return 0
    lax.fori_loop(0, _DPT // 16, fill_zero, 0)

    pltpu.sync_copy(zbuf, deg_sh.at[pl.ds(sid * _DPT, _DPT)])
    plsc.subcore_barrier()

    base = wid * _EPW

    def chunk(j, _):
        pltpu.sync_copy(dst_hbm.at[pl.ds(base + j * _CH, _CH)], dst_v)
        pltpu.sync_copy(ones_v, deg_sh.at[dst_v], add=True)
        return 0
    lax.fori_loop(0, _NCH, chunk, 0)

    plsc.subcore_barrier()
    pltpu.sync_copy(deg_sh.at[pl.ds(sid * _DPT, _DPT)],
                    deg_out.at[cid, pl.ds(sid * _DPT, _DPT)])


@functools.partial(
    pl.kernel,
    mesh=_mesh,
    out_type=jax.ShapeDtypeStruct((_NSP, _NC, _ACCN, _DIM), jnp.float32),
    scratch_types=[
        pltpu.VMEM((_CH,), jnp.int32),
        pltpu.VMEM((_CH,), jnp.int32),
        pltpu.VMEM((_CH,), jnp.int32),
        pltpu.VMEM((_CH,), jnp.int32),
        pltpu.VMEM((_CH,), jnp.int32),
        pltpu.VMEM((_CH, _DIM), jnp.float32),
        pltpu.VMEM((_CH, _DIM), jnp.float32),
        pltpu.VMEM((_APT // 4, _DIM), jnp.float32),
        pltpu.SemaphoreType.DMA,
        pltpu.SemaphoreType.DMA,
        pltpu.VMEM_SHARED((_ACCN, _DIM), jnp.float32),
    ],
    compiler_params=pltpu.CompilerParams(use_tc_tiling_on_sc=False),
)
def _sc_agg(src_hbm, dst_hbm, h2p_hbm, acc_out,
            src_v0, src_v1, dstr_v0, dstr_v1, dstl_v,
            rows_v0, rows_v1, zb2, sem0, sem1, acc_sh):
    cid = lax.axis_index("c")
    sid = lax.axis_index("s")
    wid = sid * _NC + cid
    base = wid * _EPW
    zrows = _APT // 4

    def fill_zero(i, _):
        r = i // 4
        c = (i % 4) * 16
        zb2[r, pl.ds(c, 16)] = jnp.zeros((16,), jnp.float32)
        return 0
    lax.fori_loop(0, zrows * 4, fill_zero, 0)

    for sp in range(_NSP):
        for k in range(4):
            pltpu.sync_copy(zb2, acc_sh.at[pl.ds(sid * _APT + k * zrows, zrows)])
        plsc.subcore_barrier()
        lo = sp * _SPAN
        srcs = (src_v0, src_v1)
        dstrs = (dstr_v0, dstr_v1)
        rows = (rows_v0, rows_v1)
        sems = (sem0, sem1)

        def issue(j, b):
            pltpu.sync_copy(src_hbm.at[pl.ds(base + j * _CH, _CH)], srcs[b])
            pltpu.sync_copy(dst_hbm.at[pl.ds(base + j * _CH, _CH)], dstrs[b])
            pltpu.make_async_copy(h2p_hbm.at[srcs[b]], rows[b], sems[b]).start()

        def finish(j, b):
            def remap(k2, _2):
                d = dstrs[b][pl.ds(k2 * 16, 16)]
                dl = d - lo
                ok = (dl >= 0) & (dl < _SPAN)
                dstl_v[pl.ds(k2 * 16, 16)] = jnp.where(ok, dl, _TRASH)
                return 0
            lax.fori_loop(0, _CH // 16, remap, 0)
            pltpu.make_async_copy(h2p_hbm.at[srcs[b]], rows[b], sems[b]).wait()
            pltpu.sync_copy(rows[b], acc_sh.at[dstl_v], add=True)

        issue(0, 0)

        def pair(p, _):
            for b in range(2):
                j = p * 2 + b

                @pl.when(j + 1 < _NCH)
                def _():
                    issue(j + 1, 1 - b)
                finish(j, b)
            return 0
        lax.fori_loop(0, _NCH // 2, pair, 0)

        plsc.subcore_barrier()
        pltpu.sync_copy(acc_sh.at[pl.ds(sid * _APT, _APT)],
                        acc_out.at[sp, cid, pl.ds(sid * _APT, _APT)])
        plsc.subcore_barrier()


def _tc1_body(x_ref, w0_ref, b0_ref, wc_ref, o_ref):
    h = jnp.dot(x_ref[...], w0_ref[...], preferred_element_type=jnp.float32)
    h = jnp.maximum(h + b0_ref[...][None, :], 0.0)
    o_ref[...] = jnp.dot(h, wc_ref[...], preferred_element_type=jnp.float32)


_tc1 = pl.pallas_call(
    _tc1_body,
    grid=(50,),
    in_specs=[
        pl.BlockSpec((1000, _DIN), lambda i: (i, 0)),
        pl.BlockSpec((_DIN, _DIM), lambda i: (0, 0)),
        pl.BlockSpec((_DIM,), lambda i: (0,)),
        pl.BlockSpec((_DIM, _DIM), lambda i: (0, 0)),
    ],
    out_specs=pl.BlockSpec((1000, _DIM), lambda i: (i, 0)),
    out_shape=jax.ShapeDtypeStruct((_N, _DIM), jnp.float32),
)


def _tc2_body(degp_ref, h2_ref, dinv_ref, h2p_ref):
    deg = degp_ref[0, :] + degp_ref[1, :] + 1.0
    dinv = lax.rsqrt(deg)
    dinv_ref[...] = dinv[:, None]
    h2p_ref[...] = h2_ref[...] * dinv[:, None]


_tc2 = pl.pallas_call(
    _tc2_body,
    grid=(49,),
    in_specs=[
        pl.BlockSpec((_NC, 1024), lambda i: (0, i)),
        pl.BlockSpec((1024, _DIM), lambda i: (i, 0)),
    ],
    out_specs=[
        pl.BlockSpec((1024, 1), lambda i: (i, 0)),
        pl.BlockSpec((1024, _DIM), lambda i: (i, 0)),
    ],
    out_shape=[
        jax.ShapeDtypeStruct((_N, 1), jnp.float32),
        jax.ShapeDtypeStruct((_N, _DIM), jnp.float32),
    ],
)


def _tc3_body(accs_ref, dinv_ref, h2_ref, bc_ref, o_ref):
    agg = accs_ref[0, 0] + accs_ref[0, 1]
    dinv = dinv_ref[...]
    out = jnp.maximum(
        dinv * (agg + dinv * h2_ref[...]) + bc_ref[...][None, :], 0.0)
    row = (pl.program_id(0) * 1152
           + lax.broadcasted_iota(jnp.int32, (1152, 1), 0))
    out = jnp.where(row < _N, out, 0.0)
    o_ref[...] = out.T


_tc3 = pl.pallas_call(
    _tc3_body,
    grid=(44,),
    in_specs=[
        pl.BlockSpec((1, _NC, 1152, _DIM), lambda i: (i // 16, 0, i % 16, 0)),
        pl.BlockSpec((1152, 1), lambda i: (i, 0)),
        pl.BlockSpec((1152, _DIM), lambda i: (i, 0)),
        pl.BlockSpec((_DIM,), lambda i: (0,)),
    ],
    out_specs=pl.BlockSpec((_DIM, 1152), lambda i: (0, i)),
    out_shape=jax.ShapeDtypeStruct((_DIM, _NT), jnp.float32),
)


_NB = 27
_BS = 2048


def _tc4_body(ht_ref, batch_ref, wih_ref, bih_ref, whh_ref, bhh_ref,
              w1_ref, b1_ref, w2_ref, b2_ref, o_ref):
    iota_g = lax.broadcasted_iota(jnp.int32, (_B, 1), 0)
    hs = jnp.zeros((_B, _DIM), jnp.float32)
    cs = jnp.zeros((_B, _DIM), jnp.float32)
    q_star = jnp.zeros((_B, 2 * _DIM), jnp.float32)

    for _ in range(3):
        gates = (jnp.dot(q_star, wih_ref[...], preferred_element_type=jnp.float32)
                 + bih_ref[...][None, :]
                 + jnp.dot(hs, whh_ref[...], preferred_element_type=jnp.float32)
                 + bhh_ref[...][None, :])
        i_g = jax.nn.sigmoid(gates[:, :_DIM])
        f_g = jax.nn.sigmoid(gates[:, _DIM:2 * _DIM])
        g_g = jnp.tanh(gates[:, 2 * _DIM:3 * _DIM])
        o_g = jax.nn.sigmoid(gates[:, 3 * _DIM:])
        cs = f_g * cs + i_g * g_g
        hs = o_g * jnp.tanh(cs)
        q = hs

        def pass1(nb, emax):
            bb = batch_ref[pl.ds(nb * _BS, _BS)]
            valid = (bb < _B)[None, :]
            hbt = jnp.where(valid, ht_ref[:, pl.ds(nb * _BS, _BS)], 0.0)
            obt = (iota_g == bb[None, :]).astype(jnp.float32)
            qbt = lax.dot_general(q, obt, (((0,), (0,)), ((), ())),
                                  preferred_element_type=jnp.float32)
            ebt = jnp.sum(hbt * qbt, axis=0, keepdims=True)
            m = jnp.max(jnp.where(obt > 0, ebt, -1e30), axis=1, keepdims=True)
            return jnp.maximum(emax, m)
        emax = lax.fori_loop(0, _NB, pass1,
                             jnp.full((_B, 1), -1e30, jnp.float32))

        def pass2(nb, carry):
            denom, racc = carry
            bb = batch_ref[pl.ds(nb * _BS, _BS)]
            valid = (bb < _B)[None, :]
            hbt = jnp.where(valid, ht_ref[:, pl.ds(nb * _BS, _BS)], 0.0)
            obt = (iota_g == bb[None, :]).astype(jnp.float32)
            qbt = lax.dot_general(q, obt, (((0,), (0,)), ((), ())),
                                  preferred_element_type=jnp.float32)
            ebt = jnp.sum(hbt * qbt, axis=0, keepdims=True)
            mg = lax.dot_general(emax, obt, (((0,), (0,)), ((), ())),
                                 preferred_element_type=jnp.float32)
            ab = jnp.where(obt > 0, jnp.exp(ebt - mg), 0.0)
            denom = denom + jnp.sum(ab, axis=1, keepdims=True)
            racc = racc + lax.dot_general(
                ab, hbt, (((1,), (1,)), ((), ())),
                preferred_element_type=jnp.float32)
            return denom, racc
        denom, racc = lax.fori_loop(
            0, _NB, pass2,
            (jnp.zeros((_B, 1), jnp.float32),
             jnp.zeros((_B, _DIM), jnp.float32)))

        r = jnp.where(denom > 0, racc / denom, 0.0)
        q_star = jnp.concatenate([q, r], axis=1)

    o1 = jnp.maximum(
        jnp.dot(q_star, w1_ref[...], preferred_element_type=jnp.float32)
        + b1_ref[...][None, :], 0.0)
    o_ref[...] = (jnp.dot(o1, w2_ref[...], preferred_element_type=jnp.float32)
                  + b2_ref[...][None, :])


_tc4 = pl.pallas_call(
    _tc4_body,
    out_shape=jax.ShapeDtypeStruct((_B, _OUT), jnp.float32),
)


def kernel(x, edge_index, batch, W0, b0, Wc, bc, Wih, bih, Whh, bhh,
           W1, b1, W2, b2):
    src = edge_index[0].astype(jnp.int32)
    dst = edge_index[1].astype(jnp.int32)
    pad = _E_PAD - _E
    src_p = jnp.concatenate([src, jnp.zeros((pad,), jnp.int32)])
    dst_p = jnp.concatenate([dst, jnp.full((pad,), _N, jnp.int32)])
    batch32 = jnp.concatenate([batch.astype(jnp.int32),
                           jnp.full((_NT - _N,), _B, jnp.int32)])

    h2 = _tc1(x, W0, b0, Wc)
    degp = _sc_deg(dst_p)
    dinv, h2p = _tc2(degp, h2)
    accs = _sc_agg(src_p, dst_p, h2p)
    out2 = _tc3(accs, dinv, h2, bc)
    return _tc4(out2, batch32, Wih, bih, Whh, bhh, W1, b1, W2, b2)


# speedup vs baseline: 8.5153x; 1.0039x over previous
import functools

import jax
import jax.numpy as jnp
from jax import lax
from jax.experimental import pallas as pl
from jax.experimental.pallas import tpu as pltpu
from jax.experimental.pallas import tpu_sc as plsc

_N = 50000
_E = 800000
_DIN = 128
_DIM = 64
_OUT = 2
_B = 256

_NC = 2
_NS = 16
_NW = _NC * _NS
_E_PAD = 819200
_EPW = _E_PAD // _NW
_CH = 128
_NCH = _EPW // _CH

_DEGN = 51200
_DPT = _DEGN // _NS

_NSP = 3
_SPAN = 18432
_NT = _NSP * _SPAN
_ACCN = 18496
_APT = _ACCN // _NS
_TRASH = 18432

_mesh = plsc.VectorSubcoreMesh(core_axis_name="c", subcore_axis_name="s")


@functools.partial(
    pl.kernel,
    mesh=_mesh,
    out_type=jax.ShapeDtypeStruct((_NC, _DEGN), jnp.float32),
    scratch_types=[
        pltpu.VMEM((_CH,), jnp.int32),
        pltpu.VMEM((_CH,), jnp.float32),
        pltpu.VMEM((_DPT,), jnp.float32),
        pltpu.VMEM_SHARED((_DEGN,), jnp.float32),
    ],
)
def _sc_deg(dst_hbm, deg_out, dst_v, ones_v, zbuf, deg_sh):
    cid = lax.axis_index("c")
    sid = lax.axis_index("s")
    wid = sid * _NC + cid

    def fill_ones(i, _):
        ones_v[pl.ds(i * 16, 16)] = jnp.full((16,), 1.0, jnp.float32)
        return 0
    lax.fori_loop(0, _CH // 16, fill_ones, 0)

    def fill_zero(i, _):
        zbuf[pl.ds(i * 16, 16)] = jnp.zeros((16,), jnp.float32)
        return 0
    lax.fori_loop(0, _DPT // 16, fill_zero, 0)

    pltpu.sync_copy(zbuf, deg_sh.at[pl.ds(sid * _DPT, _DPT)])
    plsc.subcore_barrier()

    base = wid * _EPW

    def chunk(j, _):
        pltpu.sync_copy(dst_hbm.at[pl.ds(base + j * _CH, _CH)], dst_v)
        pltpu.sync_copy(ones_v, deg_sh.at[dst_v], add=True)
        return 0
    lax.fori_loop(0, _NCH, chunk, 0)

    plsc.subcore_barrier()
    pltpu.sync_copy(deg_sh.at[pl.ds(sid * _DPT, _DPT)],
                    deg_out.at[cid, pl.ds(sid * _DPT, _DPT)])


@functools.partial(
    pl.kernel,
    mesh=_mesh,
    out_type=jax.ShapeDtypeStruct((_NSP, _NC, _ACCN, _DIM), jnp.float32),
    scratch_types=[
        [pltpu.VMEM((_CH,), jnp.int32) for _ in range(4)],
        [pltpu.VMEM((_CH,), jnp.int32) for _ in range(4)],
        pltpu.VMEM((_CH,), jnp.int32),
        [pltpu.VMEM((_CH, _DIM), jnp.float32) for _ in range(4)],
        pltpu.VMEM((_APT // 4, _DIM), jnp.float32),
        [pltpu.SemaphoreType.DMA for _ in range(4)],
        pltpu.VMEM_SHARED((_ACCN, _DIM), jnp.float32),
    ],
    compiler_params=pltpu.CompilerParams(use_tc_tiling_on_sc=False),
)
def _sc_agg(src_hbm, dst_hbm, h2p_hbm, acc_out,
            srcs, dstrs, dstl_v, rows, zb2, sems, acc_sh):
    cid = lax.axis_index("c")
    sid = lax.axis_index("s")
    wid = sid * _NC + cid
    base = wid * _EPW
    zrows = _APT // 4

    def fill_zero(i, _):
        r = i // 4
        c = (i % 4) * 16
        zb2[r, pl.ds(c, 16)] = jnp.zeros((16,), jnp.float32)
        return 0
    lax.fori_loop(0, zrows * 4, fill_zero, 0)

    for sp in range(_NSP):
        for k in range(4):
            pltpu.sync_copy(zb2, acc_sh.at[pl.ds(sid * _APT + k * zrows, zrows)])
        plsc.subcore_barrier()
        lo = sp * _SPAN

        def issue(j, b):
            pltpu.sync_copy(src_hbm.at[pl.ds(base + j * _CH, _CH)], srcs[b])
            pltpu.sync_copy(dst_hbm.at[pl.ds(base + j * _CH, _CH)], dstrs[b])
            pltpu.make_async_copy(h2p_hbm.at[srcs[b]], rows[b], sems[b]).start()

        def finish(j, b):
            def remap(k2, _2):
                d = dstrs[b][pl.ds(k2 * 16, 16)]
                dl = d - lo
                ok = (dl >= 0) & (dl < _SPAN)
                dstl_v[pl.ds(k2 * 16, 16)] = jnp.where(ok, dl, _TRASH)
                return 0
            lax.fori_loop(0, _CH // 16, remap, 0)
            pltpu.make_async_copy(h2p_hbm.at[srcs[b]], rows[b], sems[b]).wait()
            pltpu.sync_copy(rows[b], acc_sh.at[dstl_v], add=True)

        for b0 in range(3):
            issue(b0, b0)

        def quad(p, _):
            for b in range(4):
                j = p * 4 + b

                @pl.when(j + 3 < _NCH)
                def _():
                    issue(j + 3, (b + 3) % 4)
                finish(j, b)
            return 0
        lax.fori_loop(0, _NCH // 4, quad, 0)

        plsc.subcore_barrier()
        pltpu.sync_copy(acc_sh.at[pl.ds(sid * _APT, _APT)],
                        acc_out.at[sp, cid, pl.ds(sid * _APT, _APT)])
        plsc.subcore_barrier()


def _tc1_body(x_ref, w0_ref, b0_ref, wc_ref, o_ref):
    h = jnp.dot(x_ref[...], w0_ref[...], preferred_element_type=jnp.float32)
    h = jnp.maximum(h + b0_ref[...][None, :], 0.0)
    o_ref[...] = jnp.dot(h, wc_ref[...], preferred_element_type=jnp.float32)


_tc1 = pl.pallas_call(
    _tc1_body,
    grid=(50,),
    in_specs=[
        pl.BlockSpec((1000, _DIN), lambda i: (i, 0)),
        pl.BlockSpec((_DIN, _DIM), lambda i: (0, 0)),
        pl.BlockSpec((_DIM,), lambda i: (0,)),
        pl.BlockSpec((_DIM, _DIM), lambda i: (0, 0)),
    ],
    out_specs=pl.BlockSpec((1000, _DIM), lambda i: (i, 0)),
    out_shape=jax.ShapeDtypeStruct((_N, _DIM), jnp.float32),
)


def _tc2_body(degp_ref, h2_ref, dinv_ref, h2p_ref):
    deg = degp_ref[0, :] + degp_ref[1, :] + 1.0
    dinv = lax.rsqrt(deg)
    dinv_ref[...] = dinv[:, None]
    h2p_ref[...] = h2_ref[...] * dinv[:, None]


_tc2 = pl.pallas_call(
    _tc2_body,
    grid=(49,),
    in_specs=[
        pl.BlockSpec((_NC, 1024), lambda i: (0, i)),
        pl.BlockSpec((1024, _DIM), lambda i: (i, 0)),
    ],
    out_specs=[
        pl.BlockSpec((1024, 1), lambda i: (i, 0)),
        pl.BlockSpec((1024, _DIM), lambda i: (i, 0)),
    ],
    out_shape=[
        jax.ShapeDtypeStruct((_N, 1), jnp.float32),
        jax.ShapeDtypeStruct((_N, _DIM), jnp.float32),
    ],
)


def _tc3_body(accs_ref, dinv_ref, h2_ref, bc_ref, o_ref):
    agg = accs_ref[0, 0] + accs_ref[0, 1]
    dinv = dinv_ref[...]
    out = jnp.maximum(
        dinv * (agg + dinv * h2_ref[...]) + bc_ref[...][None, :], 0.0)
    row = (pl.program_id(0) * 1152
           + lax.broadcasted_iota(jnp.int32, (1152, 1), 0))
    out = jnp.where(row < _N, out, 0.0)
    o_ref[...] = out.T


_tc3 = pl.pallas_call(
    _tc3_body,
    grid=(44,),
    in_specs=[
        pl.BlockSpec((1, _NC, 1152, _DIM), lambda i: (i // 16, 0, i % 16, 0)),
        pl.BlockSpec((1152, 1), lambda i: (i, 0)),
        pl.BlockSpec((1152, _DIM), lambda i: (i, 0)),
        pl.BlockSpec((_DIM,), lambda i: (0,)),
    ],
    out_specs=pl.BlockSpec((_DIM, 1152), lambda i: (0, i)),
    out_shape=jax.ShapeDtypeStruct((_DIM, _NT), jnp.float32),
)


_NB = 27
_BS = 2048


def _tc4_body(ht_ref, batch_ref, wih_ref, bih_ref, whh_ref, bhh_ref,
              w1_ref, b1_ref, w2_ref, b2_ref, o_ref):
    iota_g = lax.broadcasted_iota(jnp.int32, (_B, 1), 0)
    hs = jnp.zeros((_B, _DIM), jnp.float32)
    cs = jnp.zeros((_B, _DIM), jnp.float32)
    q_star = jnp.zeros((_B, 2 * _DIM), jnp.float32)

    for _ in range(3):
        gates = (jnp.dot(q_star, wih_ref[...], preferred_element_type=jnp.float32)
                 + bih_ref[...][None, :]
                 + jnp.dot(hs, whh_ref[...], preferred_element_type=jnp.float32)
                 + bhh_ref[...][None, :])
        i_g = jax.nn.sigmoid(gates[:, :_DIM])
        f_g = jax.nn.sigmoid(gates[:, _DIM:2 * _DIM])
        g_g = jnp.tanh(gates[:, 2 * _DIM:3 * _DIM])
        o_g = jax.nn.sigmoid(gates[:, 3 * _DIM:])
        cs = f_g * cs + i_g * g_g
        hs = o_g * jnp.tanh(cs)
        q = hs

        def pass1(nb, emax):
            bb = batch_ref[pl.ds(nb * _BS, _BS)]
            valid = (bb < _B)[None, :]
            hbt = jnp.where(valid, ht_ref[:, pl.ds(nb * _BS, _BS)], 0.0)
            obt = (iota_g == bb[None, :]).astype(jnp.float32)
            qbt = lax.dot_general(q, obt, (((0,), (0,)), ((), ())),
                                  preferred_element_type=jnp.float32)
            ebt = jnp.sum(hbt * qbt, axis=0, keepdims=True)
            m = jnp.max(jnp.where(obt > 0, ebt, -1e30), axis=1, keepdims=True)
            return jnp.maximum(emax, m)
        emax = lax.fori_loop(0, _NB, pass1,
                             jnp.full((_B, 1), -1e30, jnp.float32))

        def pass2(nb, carry):
            denom, racc = carry
            bb = batch_ref[pl.ds(nb * _BS, _BS)]
            valid = (bb < _B)[None, :]
            hbt = jnp.where(valid, ht_ref[:, pl.ds(nb * _BS, _BS)], 0.0)
            obt = (iota_g == bb[None, :]).astype(jnp.float32)
            qbt = lax.dot_general(q, obt, (((0,), (0,)), ((), ())),
                                  preferred_element_type=jnp.float32)
            ebt = jnp.sum(hbt * qbt, axis=0, keepdims=True)
            mg = lax.dot_general(emax, obt, (((0,), (0,)), ((), ())),
                                 preferred_element_type=jnp.float32)
            ab = jnp.where(obt > 0, jnp.exp(ebt - mg), 0.0)
            denom = denom + jnp.sum(ab, axis=1, keepdims=True)
            racc = racc + lax.dot_general(
                ab, hbt, (((1,), (1,)), ((), ())),
                preferred_element_type=jnp.float32)
            return denom, racc
        denom, racc = lax.fori_loop(
            0, _NB, pass2,
            (jnp.zeros((_B, 1), jnp.float32),
             jnp.zeros((_B, _DIM), jnp.float32)))

        r = jnp.where(denom > 0, racc / denom, 0.0)
        q_star = jnp.concatenate([q, r], axis=1)

    o1 = jnp.maximum(
        jnp.dot(q_star, w1_ref[...], preferred_element_type=jnp.float32)
        + b1_ref[...][None, :], 0.0)
    o_ref[...] = (jnp.dot(o1, w2_ref[...], preferred_element_type=jnp.float32)
                  + b2_ref[...][None, :])


_tc4 = pl.pallas_call(
    _tc4_body,
    out_shape=jax.ShapeDtypeStruct((_B, _OUT), jnp.float32),
)


def kernel(x, edge_index, batch, W0, b0, Wc, bc, Wih, bih, Whh, bhh,
           W1, b1, W2, b2):
    src = edge_index[0].astype(jnp.int32)
    dst = edge_index[1].astype(jnp.int32)
    pad = _E_PAD - _E
    src_p = jnp.concatenate([src, jnp.zeros((pad,), jnp.int32)])
    dst_p = jnp.concatenate([dst, jnp.full((pad,), _N, jnp.int32)])
    batch32 = jnp.concatenate([batch.astype(jnp.int32),
                           jnp.full((_NT - _N,), _B, jnp.int32)])

    h2 = _tc1(x, W0, b0, Wc)
    degp = _sc_deg(dst_p)
    dinv, h2p = _tc2(degp, h2)
    accs = _sc_agg(src_p, dst_p, h2p)
    out2 = _tc3(accs, dinv, h2, bc)
    return _tc4(out2, batch32, Wih, bih, Whh, bhh, W1, b1, W2, b2)
